# baseline (speedup 1.0000x reference)
import functools

import jax
import jax.numpy as jnp
from jax import lax
from jax.experimental import pallas as pl
from jax.experimental.pallas import tpu as pltpu
from jax.experimental.pallas import tpu_sc as plsc

_B = 16384
_NC = 2
_NS = 16
_NW = _NC * _NS
_BPW = _B // _NW
_CH = 128
_NCH = _BPW // _CH
_FG = 32
_FM = 128


def _sc_gather_body(user_hbm, item_hbm, eug, eig, eum, eim,
                    gmf_out, mu_out, mi_out,
                    idx_u, idx_i, gu_v, gi_v, m_v, sem):
    wid = lax.axis_index("s") * _NC + lax.axis_index("c")
    pltpu.sync_copy(user_hbm.at[wid], idx_u)
    pltpu.sync_copy(item_hbm.at[wid], idx_i)
    descs = []
    for c in range(_NCH):
        descs.append(pltpu.async_copy(eug.at[idx_u.at[c]], gu_v.at[c], sem))
        descs.append(pltpu.async_copy(eig.at[idx_i.at[c]], gi_v.at[c], sem))
        descs.append(pltpu.async_copy(eum.at[idx_u.at[c]], m_v.at[c], sem))
    for d in descs:
        d.wait()

    def mul_row(r, _):
        for c in range(_NCH):
            for j in range(_FG // 16):
                sl = pl.ds(j * 16, 16)
                gu_v[c, r, sl] = gu_v[c, r, sl] * gi_v[c, r, sl]
        return _

    lax.fori_loop(0, _CH, mul_row, 0)

    pltpu.sync_copy(gu_v, gmf_out.at[wid])
    pltpu.sync_copy(m_v, mu_out.at[wid])
    descs = [pltpu.async_copy(eim.at[idx_i.at[c]], m_v.at[c], sem)
             for c in range(_NCH)]
    for d in descs:
        d.wait()
    pltpu.sync_copy(m_v, mi_out.at[wid])


def _sc_gather(user, item, eug, eig, eum, eim):
    mesh = plsc.VectorSubcoreMesh(core_axis_name="c", subcore_axis_name="s",
                                  num_cores=_NC, num_subcores=_NS)
    k = functools.partial(
        pl.kernel, mesh=mesh,
        compiler_params=pltpu.CompilerParams(use_tc_tiling_on_sc=False),
        out_type=(
            jax.ShapeDtypeStruct((_NW, _NCH, _CH, _FG), jnp.float32),
            jax.ShapeDtypeStruct((_NW, _NCH, _CH, _FM), jnp.float32),
            jax.ShapeDtypeStruct((_NW, _NCH, _CH, _FM), jnp.float32),
        ),
        scratch_types=[
            pltpu.VMEM((_NCH, _CH), jnp.int32),
            pltpu.VMEM((_NCH, _CH), jnp.int32),
            pltpu.VMEM((_NCH, _CH, _FG), jnp.float32),
            pltpu.VMEM((_NCH, _CH, _FG), jnp.float32),
            pltpu.VMEM((_NCH, _CH, _FM), jnp.float32),
            pltpu.SemaphoreType.DMA,
        ],
    )(_sc_gather_body)
    u3 = user.reshape(_NW, _NCH, _CH)
    i3 = item.reshape(_NW, _NCH, _CH)
    gmf, mu, mi = k(u3, i3, eug, eig, eum, eim)
    return (gmf.reshape(_B, _FG), mu.reshape(_B, _FM), mi.reshape(_B, _FM))


_BLK = 2048


def _mlp_body(mu_ref, mi_ref, gmf_ref, w1a, w1b, b1, w2, b2, w3, b3,
              wpa, wpb, bp, out_ref):
    x = jnp.dot(mu_ref[...], w1a[...], preferred_element_type=jnp.float32)
    x = x + jnp.dot(mi_ref[...], w1b[...], preferred_element_type=jnp.float32)
    x = jnp.maximum(x + b1[...], 0.0)
    x = jnp.maximum(jnp.dot(x, w2[...], preferred_element_type=jnp.float32)
                    + b2[...], 0.0)
    x = jnp.maximum(jnp.dot(x, w3[...], preferred_element_type=jnp.float32)
                    + b3[...], 0.0)
    out = jnp.dot(gmf_ref[...], wpa[...], preferred_element_type=jnp.float32)
    out = out + jnp.dot(x, wpb[...], preferred_element_type=jnp.float32)
    out_ref[...] = out + bp[...]


def _tc_mlp(mu, mi, gmf, W1, b1, W2, b2, W3, b3, Wp, bp):
    full = lambda shape: pl.BlockSpec(shape, lambda i: (0, 0))
    grid = (_B // _BLK,)
    return pl.pallas_call(
        _mlp_body,
        grid=grid,
        in_specs=[
            pl.BlockSpec((_BLK, _FM), lambda i: (i, 0)),
            pl.BlockSpec((_BLK, _FM), lambda i: (i, 0)),
            pl.BlockSpec((_BLK, _FG), lambda i: (i, 0)),
            full((128, 128)), full((128, 128)), full((1, 128)),
            full((128, 64)), full((1, 64)),
            full((64, 32)), full((1, 32)),
            full((32, 1)), full((32, 1)), full((1, 1)),
        ],
        out_specs=pl.BlockSpec((_BLK, 1), lambda i: (i, 0)),
        out_shape=jax.ShapeDtypeStruct((_B, 1), jnp.float32),
    )(mu, mi, gmf,
      W1[:128], W1[128:], b1.reshape(1, -1),
      W2, b2.reshape(1, -1), W3, b3.reshape(1, -1),
      Wp[:32], Wp[32:], bp.reshape(1, 1))


def kernel(user, item, embed_user_GMF, embed_item_GMF, embed_user_MLP,
           embed_item_MLP, W1, b1, W2, b2, W3, b3, Wp, bp):
    user = user.astype(jnp.int32)
    item = item.astype(jnp.int32)
    gmf, mu, mi = _sc_gather(user, item, embed_user_GMF, embed_item_GMF,
                             embed_user_MLP, embed_item_MLP)
    out = _tc_mlp(mu, mi, gmf, W1, b1, W2, b2, W3, b3, Wp, bp)
    return out.reshape(-1)
